# baseline (speedup 1.0000x reference)
import functools

import jax
import jax.numpy as jnp
from jax import lax
from jax.experimental import pallas as pl
from jax.experimental.pallas import tpu as pltpu
from jax.experimental.pallas import tpu_sc as plsc

_BATCH = 16384
_DIM = 64
_NC = 2
_NS = 16
_NW = _NC * _NS
_PER_W = _BATCH // _NW
_CHUNK = 64
_NCHUNK = _PER_W // _CHUNK
_MARGIN = 1.0


def _vsqrt(x):
    i = lax.bitcast_convert_type(x, jnp.int32)
    y = lax.bitcast_convert_type(jnp.int32(0x5F3759DF) - (i >> 1), jnp.float32)
    xh = x * 0.5
    y = y * (1.5 - xh * y * y)
    y = y * (1.5 - xh * y * y)
    y = y * (1.5 - xh * y * y)
    return x * y


def _pair_table(table_t, n_rows, eb):
    grid = (n_rows + eb - 1) // eb

    def body(in_ref, eye_ref, out_ref):
        u = in_ref[...].astype(jnp.bfloat16)
        t = lax.dot_general(u, eye_ref[...], (((0,), (0,)), ((), ())),
                            preferred_element_type=jnp.float32)
        bands = [
            jnp.concatenate([t[128 * b: 128 * b + 64],
                             t[128 * b + 64: 128 * b + 128]], axis=1)
            for b in range(eb // 128)
        ]
        out_ref[...] = jnp.concatenate(bands, axis=0)

    eye = jnp.eye(_DIM, dtype=jnp.bfloat16)
    return pl.pallas_call(
        body,
        grid=(grid,),
        in_specs=[pl.BlockSpec((_DIM, eb), lambda i: (0, i)),
                  pl.BlockSpec((_DIM, _DIM), lambda i: (0, 0))],
        out_specs=pl.BlockSpec((eb // 2, 128), lambda i: (i, 0)),
        out_shape=jax.ShapeDtypeStruct((grid * (eb // 2), 128), jnp.float32),
    )(table_t, eye)


def _make_sc_call(interpret=False):
    mesh = plsc.VectorSubcoreMesh(
        core_axis_name="c", subcore_axis_name="s", num_cores=_NC, num_subcores=_NS
    )
    idxF_t = pltpu.VMEM((_PER_W,), jnp.int32)
    gl_t = pltpu.VMEM((_CHUNK,), jnp.int32)
    row_t = pltpu.VMEM((_CHUNK, 128), jnp.float32)

    @functools.partial(
        pl.kernel,
        mesh=mesh,
        out_type=jax.ShapeDtypeStruct((_NW, 128), jnp.float32),
        scratch_types=[
            idxF_t, idxF_t, idxF_t, idxF_t, idxF_t, idxF_t,
            gl_t, gl_t, gl_t, gl_t,
            row_t, row_t, row_t, row_t,
            pltpu.VMEM((512, 128), jnp.float32),
            pltpu.VMEM((128,), jnp.float32),
            pltpu.SemaphoreType.DMA,
        ],
        compiler_params=pltpu.CompilerParams(
            needs_layout_passes=False, use_tc_tiling_on_sc=True
        ),
        interpret=interpret,
    )
    def sc_call(ph, pr, pt, nh, nr, nt, ent2, rel2, out,
                ph_i, pr_i, pt_i, nh_i, nr_i, nt_i,
                gl_ph, gl_pt, gl_nh, gl_nt,
                ph_r, pt_r, nh_r, nt_r, rel_v, acc_v, sem):
        wid = lax.axis_index("s") * _NC + lax.axis_index("c")
        base = wid * _PER_W
        lane = lax.iota(jnp.int32, 16)

        i1 = pltpu.async_copy(ph.at[pl.ds(base, _PER_W)], ph_i, sem)
        i2 = pltpu.async_copy(pr.at[pl.ds(base, _PER_W)], pr_i, sem)
        i3 = pltpu.async_copy(pt.at[pl.ds(base, _PER_W)], pt_i, sem)
        i4 = pltpu.async_copy(nh.at[pl.ds(base, _PER_W)], nh_i, sem)
        i5 = pltpu.async_copy(nr.at[pl.ds(base, _PER_W)], nr_i, sem)
        i6 = pltpu.async_copy(nt.at[pl.ds(base, _PER_W)], nt_i, sem)
        r0 = pltpu.async_copy(rel2, rel_v, sem)
        i1.wait(); i2.wait(); i3.wait(); i4.wait(); i5.wait(); i6.wait(); r0.wait()

        def _prow(v):
            return ((v >> 7) << 6) | (v & 63)

        def chunk_body(ci, acc):
            off = ci * _CHUNK
            for buf_i, gl in ((ph_i, gl_ph), (pt_i, gl_pt),
                              (nh_i, gl_nh), (nt_i, gl_nt)):
                for k in range(_CHUNK // 16):
                    sl = pl.ds(k * 16, 16)
                    gl[sl] = _prow(buf_i[pl.ds(off + k * 16, 16)])
            g1 = pltpu.async_copy(ent2.at[gl_ph], ph_r, sem)
            g3 = pltpu.async_copy(ent2.at[gl_pt], pt_r, sem)
            g4 = pltpu.async_copy(ent2.at[gl_nh], nh_r, sem)
            g6 = pltpu.async_copy(ent2.at[gl_nt], nt_r, sem)
            g1.wait(); g3.wait(); g4.wait(); g6.wait()

            def group_body(g, acc_in):
                slg = pl.ds(off + g * 16, 16)
                slots = g * 16 + lane
                c_ph = ph_i[slg] & 64
                c_pr = pr_i[slg] & 64
                c_pt = pt_i[slg] & 64
                c_nh = nh_i[slg] & 64
                c_nr = nr_i[slg] & 64
                c_nt = nt_i[slg] & 64
                r_pr = _prow(pr_i[slg])
                r_nr = _prow(nr_i[slg])
                pos_ssq = jnp.zeros((16,), jnp.float32)
                neg_ssq = jnp.zeros((16,), jnp.float32)
                for f in range(_DIM):
                    fv = (lane + f) & 63
                    d = (plsc.load_gather(ph_r, [slots, c_ph + fv])
                         + plsc.load_gather(rel_v, [r_pr, c_pr + fv])
                         - plsc.load_gather(pt_r, [slots, c_pt + fv]))
                    pos_ssq = pos_ssq + d * d
                    e = (plsc.load_gather(nh_r, [slots, c_nh + fv])
                         + plsc.load_gather(rel_v, [r_nr, c_nr + fv])
                         - plsc.load_gather(nt_r, [slots, c_nt + fv]))
                    neg_ssq = neg_ssq + e * e
                term = jnp.maximum(_MARGIN + _vsqrt(pos_ssq) - _vsqrt(neg_ssq), 0.0)
                return acc_in + term

            return lax.fori_loop(0, _CHUNK // 16, group_body, acc)

        acc = lax.fori_loop(0, _NCHUNK, chunk_body, jnp.zeros((16,), jnp.float32))
        for k in range(8):
            acc_v[pl.ds(k * 16, 16)] = acc if k == 0 else jnp.zeros((16,), jnp.float32)
        pltpu.sync_copy(acc_v, out.at[wid])

    return sc_call


_sc_call = _make_sc_call()


def kernel(pos_head, pos_relation, pos_tail, neg_head, neg_relation, neg_tail,
           entity_embedding, relation_embedding):
    ent2 = _pair_table(entity_embedding.T, 1000000, 16384)
    rel2 = _pair_table(relation_embedding.T, 1000, 1024)
    partials = _sc_call(pos_head, pos_relation, pos_tail, neg_head, neg_relation,
                        neg_tail, ent2, rel2)
    return jnp.sum(partials)


# speedup vs baseline: 1.0786x; 1.0786x over previous
import functools

import jax
import jax.numpy as jnp
from jax import lax
from jax.experimental import pallas as pl
from jax.experimental.pallas import tpu as pltpu
from jax.experimental.pallas import tpu_sc as plsc

_BATCH = 16384
_DIM = 64
_NC = 2
_NS = 16
_NW = _NC * _NS
_PER_W = _BATCH // _NW
_CHUNK = 64
_NCHUNK = _PER_W // _CHUNK
_MARGIN = 1.0


def _vsqrt(x):
    i = lax.bitcast_convert_type(x, jnp.int32)
    y = lax.bitcast_convert_type(jnp.int32(0x5F3759DF) - (i >> 1), jnp.float32)
    xh = x * 0.5
    y = y * (1.5 - xh * y * y)
    y = y * (1.5 - xh * y * y)
    y = y * (1.5 - xh * y * y)
    return x * y


def _pair_table(table_t, n_rows, eb):
    grid = (n_rows + eb - 1) // eb

    def body(in_ref, eye_ref, out_ref):
        u = in_ref[...].astype(jnp.bfloat16)
        t = lax.dot_general(u, eye_ref[...], (((0,), (0,)), ((), ())),
                            preferred_element_type=jnp.float32)
        bands = [
            jnp.concatenate([t[128 * b: 128 * b + 64],
                             t[128 * b + 64: 128 * b + 128]], axis=1)
            for b in range(eb // 128)
        ]
        out_ref[...] = jnp.concatenate(bands, axis=0)

    eye = jnp.eye(_DIM, dtype=jnp.bfloat16)
    return pl.pallas_call(
        body,
        grid=(grid,),
        in_specs=[pl.BlockSpec((_DIM, eb), lambda i: (0, i)),
                  pl.BlockSpec((_DIM, _DIM), lambda i: (0, 0))],
        out_specs=pl.BlockSpec((eb // 2, 128), lambda i: (i, 0)),
        out_shape=jax.ShapeDtypeStruct((grid * (eb // 2), 128), jnp.float32),
    )(table_t, eye)


def _make_sc_call(interpret=False):
    mesh = plsc.VectorSubcoreMesh(
        core_axis_name="c", subcore_axis_name="s", num_cores=_NC, num_subcores=_NS
    )
    idxF_t = pltpu.VMEM((_PER_W,), jnp.int32)
    gl_t = pltpu.VMEM((_CHUNK,), jnp.int32)
    row_t = pltpu.VMEM((_CHUNK, 128), jnp.float32)

    @functools.partial(
        pl.kernel,
        mesh=mesh,
        out_type=jax.ShapeDtypeStruct((_NW, 128), jnp.float32),
        scratch_types=[
            idxF_t, idxF_t, idxF_t, idxF_t, idxF_t, idxF_t,
            gl_t, gl_t, gl_t, gl_t,
            row_t, row_t, row_t, row_t,
            pltpu.VMEM((512, 128), jnp.float32),
            pltpu.VMEM((128,), jnp.float32),
            pltpu.SemaphoreType.DMA,
        ],
        compiler_params=pltpu.CompilerParams(
            needs_layout_passes=False, use_tc_tiling_on_sc=True
        ),
        interpret=interpret,
    )
    def sc_call(ph, pr, pt, nh, nr, nt, ent2, rel2, out,
                ph_i, pr_i, pt_i, nh_i, nr_i, nt_i,
                gl_ph, gl_pt, gl_nh, gl_nt,
                ph_r, pt_r, nh_r, nt_r, rel_v, acc_v, sem):
        wid = lax.axis_index("s") * _NC + lax.axis_index("c")
        base = wid * _PER_W
        lane = lax.iota(jnp.int32, 16)

        i1 = pltpu.async_copy(ph.at[pl.ds(base, _PER_W)], ph_i, sem)
        i2 = pltpu.async_copy(pr.at[pl.ds(base, _PER_W)], pr_i, sem)
        i3 = pltpu.async_copy(pt.at[pl.ds(base, _PER_W)], pt_i, sem)
        i4 = pltpu.async_copy(nh.at[pl.ds(base, _PER_W)], nh_i, sem)
        i5 = pltpu.async_copy(nr.at[pl.ds(base, _PER_W)], nr_i, sem)
        i6 = pltpu.async_copy(nt.at[pl.ds(base, _PER_W)], nt_i, sem)
        r0 = pltpu.async_copy(rel2, rel_v, sem)
        i1.wait(); i2.wait(); i3.wait(); i4.wait(); i5.wait(); i6.wait(); r0.wait()

        def _prow(v):
            return ((v >> 7) << 6) | (v & 63)

        def chunk_body(ci, acc):
            off = ci * _CHUNK
            for buf_i, gl in ((ph_i, gl_ph), (pt_i, gl_pt),
                              (nh_i, gl_nh), (nt_i, gl_nt)):
                for k in range(_CHUNK // 16):
                    sl = pl.ds(k * 16, 16)
                    gl[sl] = _prow(buf_i[pl.ds(off + k * 16, 16)])
            g1 = pltpu.async_copy(ent2.at[gl_ph], ph_r, sem)
            g3 = pltpu.async_copy(ent2.at[gl_pt], pt_r, sem)
            g4 = pltpu.async_copy(ent2.at[gl_nh], nh_r, sem)
            g6 = pltpu.async_copy(ent2.at[gl_nt], nt_r, sem)
            g1.wait(); g3.wait(); g4.wait(); g6.wait()

            def group_body(g, acc_in):
                slg = pl.ds(off + g * 16, 16)
                slots = g * 16 + lane
                c_ph = ph_i[slg] & 64
                c_pr = pr_i[slg] & 64
                c_pt = pt_i[slg] & 64
                c_nh = nh_i[slg] & 64
                c_nr = nr_i[slg] & 64
                c_nt = nt_i[slg] & 64
                r_pr = _prow(pr_i[slg])
                r_nr = _prow(nr_i[slg])
                pos_ssq = jnp.zeros((16,), jnp.float32)
                neg_ssq = jnp.zeros((16,), jnp.float32)
                for f in range(_DIM):
                    fv = (lane + f) & 63
                    d = (plsc.load_gather(ph_r, [slots, c_ph + fv])
                         + plsc.load_gather(rel_v, [r_pr, c_pr + fv])
                         - plsc.load_gather(pt_r, [slots, c_pt + fv]))
                    pos_ssq = pos_ssq + d * d
                    e = (plsc.load_gather(nh_r, [slots, c_nh + fv])
                         + plsc.load_gather(rel_v, [r_nr, c_nr + fv])
                         - plsc.load_gather(nt_r, [slots, c_nt + fv]))
                    neg_ssq = neg_ssq + e * e
                term = jnp.maximum(_MARGIN + _vsqrt(pos_ssq) - _vsqrt(neg_ssq), 0.0)
                return acc_in + term

            return lax.fori_loop(0, _CHUNK // 16, group_body, acc)

        acc = lax.fori_loop(0, _NCHUNK, chunk_body, jnp.zeros((16,), jnp.float32))
        for k in range(8):
            acc_v[pl.ds(k * 16, 16)] = acc if k == 0 else jnp.zeros((16,), jnp.float32)
        pltpu.sync_copy(acc_v, out.at[wid])

    return sc_call


_sc_call = _make_sc_call()


def kernel(pos_head, pos_relation, pos_tail, neg_head, neg_relation, neg_tail,
           entity_embedding, relation_embedding):
    ent2 = _pair_table(entity_embedding.T, 1000000, 32768)
    rel2 = _pair_table(relation_embedding.T, 1000, 1024)
    partials = _sc_call(pos_head, pos_relation, pos_tail, neg_head, neg_relation,
                        neg_tail, ent2, rel2)
    return jnp.sum(partials)


# speedup vs baseline: 1.0862x; 1.0070x over previous
import functools

import jax
import jax.numpy as jnp
from jax import lax
from jax.experimental import pallas as pl
from jax.experimental.pallas import tpu as pltpu
from jax.experimental.pallas import tpu_sc as plsc

_BATCH = 16384
_DIM = 64
_NC = 2
_NS = 16
_NW = _NC * _NS
_PER_W = _BATCH // _NW
_CHUNK = 64
_NCHUNK = _PER_W // _CHUNK
_MARGIN = 1.0


def _vsqrt(x):
    i = lax.bitcast_convert_type(x, jnp.int32)
    y = lax.bitcast_convert_type(jnp.int32(0x5F3759DF) - (i >> 1), jnp.float32)
    xh = x * 0.5
    y = y * (1.5 - xh * y * y)
    y = y * (1.5 - xh * y * y)
    y = y * (1.5 - xh * y * y)
    return x * y


def _pair_table(table_t, n_rows, eb):
    grid = (n_rows + eb - 1) // eb

    def body(in_ref, eye_ref, out_ref):
        u = in_ref[...].astype(jnp.bfloat16)
        t = lax.dot_general(u, eye_ref[...], (((0,), (0,)), ((), ())),
                            preferred_element_type=jnp.float32)
        bands = [
            jnp.concatenate([t[128 * b: 128 * b + 64],
                             t[128 * b + 64: 128 * b + 128]], axis=1)
            for b in range(eb // 128)
        ]
        out_ref[...] = jnp.concatenate(bands, axis=0)

    eye = jnp.eye(_DIM, dtype=jnp.bfloat16)
    return pl.pallas_call(
        body,
        grid=(grid,),
        in_specs=[pl.BlockSpec((_DIM, eb), lambda i: (0, i)),
                  pl.BlockSpec((_DIM, _DIM), lambda i: (0, 0))],
        out_specs=pl.BlockSpec((eb // 2, 128), lambda i: (i, 0)),
        out_shape=jax.ShapeDtypeStruct((grid * (eb // 2), 128), jnp.float32),
    )(table_t, eye)


def _make_sc_call(interpret=False):
    mesh = plsc.VectorSubcoreMesh(
        core_axis_name="c", subcore_axis_name="s", num_cores=_NC, num_subcores=_NS
    )
    idxF_t = pltpu.VMEM((_PER_W,), jnp.int32)
    gl_t = pltpu.VMEM((_CHUNK,), jnp.int32)
    row_t = pltpu.VMEM((_CHUNK, 128), jnp.float32)

    @functools.partial(
        pl.kernel,
        mesh=mesh,
        out_type=jax.ShapeDtypeStruct((_NW, 128), jnp.float32),
        scratch_types=[
            idxF_t, idxF_t, idxF_t, idxF_t, idxF_t, idxF_t,
            gl_t, gl_t, gl_t, gl_t,
            row_t, row_t, row_t, row_t,
            pltpu.VMEM((512, 128), jnp.float32),
            pltpu.VMEM((128,), jnp.float32),
            pltpu.SemaphoreType.DMA,
        ],
        compiler_params=pltpu.CompilerParams(
            needs_layout_passes=False, use_tc_tiling_on_sc=True
        ),
        interpret=interpret,
    )
    def sc_call(ph, pr, pt, nh, nr, nt, ent2, rel2, out,
                ph_i, pr_i, pt_i, nh_i, nr_i, nt_i,
                gl_ph, gl_pt, gl_nh, gl_nt,
                ph_r, pt_r, nh_r, nt_r, rel_v, acc_v, sem):
        wid = lax.axis_index("s") * _NC + lax.axis_index("c")
        base = wid * _PER_W
        lane = lax.iota(jnp.int32, 16)

        i1 = pltpu.async_copy(ph.at[pl.ds(base, _PER_W)], ph_i, sem)
        i2 = pltpu.async_copy(pr.at[pl.ds(base, _PER_W)], pr_i, sem)
        i3 = pltpu.async_copy(pt.at[pl.ds(base, _PER_W)], pt_i, sem)
        i4 = pltpu.async_copy(nh.at[pl.ds(base, _PER_W)], nh_i, sem)
        i5 = pltpu.async_copy(nr.at[pl.ds(base, _PER_W)], nr_i, sem)
        i6 = pltpu.async_copy(nt.at[pl.ds(base, _PER_W)], nt_i, sem)
        r0 = pltpu.async_copy(rel2, rel_v, sem)
        i1.wait(); i2.wait(); i3.wait(); i4.wait(); i5.wait(); i6.wait(); r0.wait()

        def _prow(v):
            return ((v >> 7) << 6) | (v & 63)

        def chunk_body(ci, acc):
            off = ci * _CHUNK
            for buf_i, gl in ((ph_i, gl_ph), (pt_i, gl_pt),
                              (nh_i, gl_nh), (nt_i, gl_nt)):
                for k in range(_CHUNK // 16):
                    sl = pl.ds(k * 16, 16)
                    gl[sl] = _prow(buf_i[pl.ds(off + k * 16, 16)])
            g1 = pltpu.async_copy(ent2.at[gl_ph], ph_r, sem)
            g3 = pltpu.async_copy(ent2.at[gl_pt], pt_r, sem)
            g4 = pltpu.async_copy(ent2.at[gl_nh], nh_r, sem)
            g6 = pltpu.async_copy(ent2.at[gl_nt], nt_r, sem)
            g1.wait(); g3.wait(); g4.wait(); g6.wait()

            def group_body(g, acc_in):
                slg = pl.ds(off + g * 16, 16)
                slots = g * 16 + lane
                c_ph = ph_i[slg] & 64
                c_pr = pr_i[slg] & 64
                c_pt = pt_i[slg] & 64
                c_nh = nh_i[slg] & 64
                c_nr = nr_i[slg] & 64
                c_nt = nt_i[slg] & 64
                r_pr = _prow(pr_i[slg])
                r_nr = _prow(nr_i[slg])
                pos_ssq = jnp.zeros((16,), jnp.float32)
                neg_ssq = jnp.zeros((16,), jnp.float32)
                for f in range(_DIM):
                    fv = (lane + f) & 63
                    d = (plsc.load_gather(ph_r, [slots, c_ph + fv])
                         + plsc.load_gather(rel_v, [r_pr, c_pr + fv])
                         - plsc.load_gather(pt_r, [slots, c_pt + fv]))
                    pos_ssq = pos_ssq + d * d
                    e = (plsc.load_gather(nh_r, [slots, c_nh + fv])
                         + plsc.load_gather(rel_v, [r_nr, c_nr + fv])
                         - plsc.load_gather(nt_r, [slots, c_nt + fv]))
                    neg_ssq = neg_ssq + e * e
                term = jnp.maximum(_MARGIN + _vsqrt(pos_ssq) - _vsqrt(neg_ssq), 0.0)
                return acc_in + term

            return lax.fori_loop(0, _CHUNK // 16, group_body, acc)

        acc = lax.fori_loop(0, _NCHUNK, chunk_body, jnp.zeros((16,), jnp.float32))
        for k in range(8):
            acc_v[pl.ds(k * 16, 16)] = acc if k == 0 else jnp.zeros((16,), jnp.float32)
        pltpu.sync_copy(acc_v, out.at[wid])

    return sc_call


_sc_call = _make_sc_call()


def kernel(pos_head, pos_relation, pos_tail, neg_head, neg_relation, neg_tail,
           entity_embedding, relation_embedding):
    ent2 = _pair_table(entity_embedding.T, 1000000, 40960)
    rel2 = _pair_table(relation_embedding.T, 1000, 1024)
    partials = _sc_call(pos_head, pos_relation, pos_tail, neg_head, neg_relation,
                        neg_tail, ent2, rel2)
    return jnp.sum(partials)


# speedup vs baseline: 1.0896x; 1.0032x over previous
import functools

import jax
import jax.numpy as jnp
from jax import lax
from jax.experimental import pallas as pl
from jax.experimental.pallas import tpu as pltpu
from jax.experimental.pallas import tpu_sc as plsc

_BATCH = 16384
_DIM = 64
_NC = 2
_NS = 16
_NW = _NC * _NS
_PER_W = _BATCH // _NW
_CHUNK = 32
_NCHUNK = _PER_W // _CHUNK
_NPAIR = _NCHUNK // 2
_MARGIN = 1.0


def _vsqrt(x):
    i = lax.bitcast_convert_type(x, jnp.int32)
    y = lax.bitcast_convert_type(jnp.int32(0x5F3759DF) - (i >> 1), jnp.float32)
    xh = x * 0.5
    y = y * (1.5 - xh * y * y)
    y = y * (1.5 - xh * y * y)
    y = y * (1.5 - xh * y * y)
    return x * y


def _pair_table(table_t, n_rows, eb):
    grid = (n_rows + eb - 1) // eb

    def body(in_ref, eye_ref, out_ref):
        u = in_ref[...].astype(jnp.bfloat16)
        t = lax.dot_general(u, eye_ref[...], (((0,), (0,)), ((), ())),
                            preferred_element_type=jnp.float32)
        bands = [
            jnp.concatenate([t[128 * b: 128 * b + 64],
                             t[128 * b + 64: 128 * b + 128]], axis=1)
            for b in range(eb // 128)
        ]
        out_ref[...] = jnp.concatenate(bands, axis=0)

    eye = jnp.eye(_DIM, dtype=jnp.bfloat16)
    return pl.pallas_call(
        body,
        grid=(grid,),
        in_specs=[pl.BlockSpec((_DIM, eb), lambda i: (0, i)),
                  pl.BlockSpec((_DIM, _DIM), lambda i: (0, 0))],
        out_specs=pl.BlockSpec((eb // 2, 128), lambda i: (i, 0)),
        out_shape=jax.ShapeDtypeStruct((grid * (eb // 2), 128), jnp.float32),
    )(table_t, eye)


def _make_sc_call(interpret=False):
    mesh = plsc.VectorSubcoreMesh(
        core_axis_name="c", subcore_axis_name="s", num_cores=_NC, num_subcores=_NS
    )
    idxF_t = pltpu.VMEM((_PER_W,), jnp.int32)
    gl_t = pltpu.VMEM((_CHUNK,), jnp.int32)
    row_t = pltpu.VMEM((_CHUNK, 128), jnp.float32)

    @functools.partial(
        pl.kernel,
        mesh=mesh,
        out_type=jax.ShapeDtypeStruct((_NW, 128), jnp.float32),
        scratch_types=[
            idxF_t, idxF_t, idxF_t, idxF_t, idxF_t, idxF_t,
            gl_t, gl_t, gl_t, gl_t,
            gl_t, gl_t, gl_t, gl_t,
            row_t, row_t, row_t, row_t,
            row_t, row_t, row_t, row_t,
            pltpu.VMEM((512, 128), jnp.float32),
            pltpu.VMEM((128,), jnp.float32),
            pltpu.SemaphoreType.DMA,
            pltpu.SemaphoreType.DMA,
            pltpu.SemaphoreType.DMA,
        ],
        compiler_params=pltpu.CompilerParams(
            needs_layout_passes=False, use_tc_tiling_on_sc=True
        ),
        interpret=interpret,
    )
    def sc_call(ph, pr, pt, nh, nr, nt, ent2, rel2, out,
                ph_i, pr_i, pt_i, nh_i, nr_i, nt_i,
                glA_ph, glA_pt, glA_nh, glA_nt,
                glB_ph, glB_pt, glB_nh, glB_nt,
                phA_r, ptA_r, nhA_r, ntA_r,
                phB_r, ptB_r, nhB_r, ntB_r,
                rel_v, acc_v, sem, semA, semB):
        wid = lax.axis_index("s") * _NC + lax.axis_index("c")
        base = wid * _PER_W
        lane = lax.iota(jnp.int32, 16)

        i1 = pltpu.async_copy(ph.at[pl.ds(base, _PER_W)], ph_i, sem)
        i2 = pltpu.async_copy(pr.at[pl.ds(base, _PER_W)], pr_i, sem)
        i3 = pltpu.async_copy(pt.at[pl.ds(base, _PER_W)], pt_i, sem)
        i4 = pltpu.async_copy(nh.at[pl.ds(base, _PER_W)], nh_i, sem)
        i5 = pltpu.async_copy(nr.at[pl.ds(base, _PER_W)], nr_i, sem)
        i6 = pltpu.async_copy(nt.at[pl.ds(base, _PER_W)], nt_i, sem)
        r0 = pltpu.async_copy(rel2, rel_v, sem)
        i1.wait(); i2.wait(); i3.wait(); i4.wait(); i5.wait(); i6.wait(); r0.wait()

        def _prow(v):
            return ((v >> 7) << 6) | (v & 63)

        sideA = ((ph_i, glA_ph, phA_r), (pt_i, glA_pt, ptA_r),
                 (nh_i, glA_nh, nhA_r), (nt_i, glA_nt, ntA_r))
        sideB = ((ph_i, glB_ph, phB_r), (pt_i, glB_pt, ptB_r),
                 (nh_i, glB_nh, nhB_r), (nt_i, glB_nt, ntB_r))

        def _issue(ci, side, s):
            off = ci * _CHUNK
            for buf_i, gl, _row in side:
                for k in range(_CHUNK // 16):
                    sl = pl.ds(k * 16, 16)
                    gl[sl] = _prow(buf_i[pl.ds(off + k * 16, 16)])
            for _buf_i, gl, row in side:
                pltpu.async_copy(ent2.at[gl], row, s)

        def _drain(side, s):
            for _buf_i, gl, row in side:
                pltpu.make_async_copy(ent2.at[gl], row, s).wait()

        def _compute(ci, side, acc_in):
            off = ci * _CHUNK
            rows4 = tuple(row for _b, _g, row in side)

            def group_body(g, a):
                slg = pl.ds(off + g * 16, 16)
                slots = g * 16 + lane
                c_ph = ph_i[slg] & 64
                c_pr = pr_i[slg] & 64
                c_pt = pt_i[slg] & 64
                c_nh = nh_i[slg] & 64
                c_nr = nr_i[slg] & 64
                c_nt = nt_i[slg] & 64
                r_pr = _prow(pr_i[slg])
                r_nr = _prow(nr_i[slg])
                rph, rpt, rnh, rnt = rows4
                pos_ssq = jnp.zeros((16,), jnp.float32)
                neg_ssq = jnp.zeros((16,), jnp.float32)
                for f in range(_DIM):
                    fv = (lane + f) & 63
                    d = (plsc.load_gather(rph, [slots, c_ph + fv])
                         + plsc.load_gather(rel_v, [r_pr, c_pr + fv])
                         - plsc.load_gather(rpt, [slots, c_pt + fv]))
                    pos_ssq = pos_ssq + d * d
                    e = (plsc.load_gather(rnh, [slots, c_nh + fv])
                         + plsc.load_gather(rel_v, [r_nr, c_nr + fv])
                         - plsc.load_gather(rnt, [slots, c_nt + fv]))
                    neg_ssq = neg_ssq + e * e
                term = jnp.maximum(_MARGIN + _vsqrt(pos_ssq) - _vsqrt(neg_ssq), 0.0)
                return a + term

            return lax.fori_loop(0, _CHUNK // 16, group_body, acc_in)

        _issue(0, sideA, semA)

        def pair_body(i, acc):
            ca = 2 * i
            _issue(ca + 1, sideB, semB)
            _drain(sideA, semA)
            acc = _compute(ca, sideA, acc)

            @pl.when(i < _NPAIR - 1)
            def _():
                _issue(ca + 2, sideA, semA)

            _drain(sideB, semB)
            return _compute(ca + 1, sideB, acc)

        acc = lax.fori_loop(0, _NPAIR, pair_body, jnp.zeros((16,), jnp.float32))
        for k in range(8):
            acc_v[pl.ds(k * 16, 16)] = acc if k == 0 else jnp.zeros((16,), jnp.float32)
        pltpu.sync_copy(acc_v, out.at[wid])

    return sc_call


_sc_call = _make_sc_call()


def kernel(pos_head, pos_relation, pos_tail, neg_head, neg_relation, neg_tail,
           entity_embedding, relation_embedding):
    ent2 = _pair_table(entity_embedding.T, 1000000, 32768)
    rel2 = _pair_table(relation_embedding.T, 1000, 1024)
    partials = _sc_call(pos_head, pos_relation, pos_tail, neg_head, neg_relation,
                        neg_tail, ent2, rel2)
    return jnp.sum(partials)
